# baseline (speedup 1.0000x reference)
import jax
import jax.numpy as jnp
from jax import lax
from jax.experimental import pallas as pl
from jax.experimental.pallas import tpu as pltpu
from jax.experimental.pallas import tpu_sc as plsc

N = 10000
E = 320000
DIN = 128
DE = 16
DOUT = 128

NC = 2
NS = 16
NW = NC * NS
CH = 64
NCHUNK = -(-E // (NW * CH))
EPW = NCHUNK * CH
EPAD = NW * EPW
RPT = 624
REM = N - NS * RPT
NZCH = -(-N // CH)


def _p1_body(nf_ref, wa_ref, ad_ref, mx_ref):
    i = pl.program_id(0)
    y = jnp.dot(nf_ref[...], wa_ref[...], preferred_element_type=jnp.float32)
    ad_ref[...] = y
    m = jnp.max(y, axis=0, keepdims=True)

    @pl.when(i == 0)
    def _():
        mx_ref[...] = m

    @pl.when(i > 0)
    def _():
        mx_ref[...] = jnp.maximum(mx_ref[...], m)


def _p2_body(ef_ref, w_ref, b_ref, ee_ref, mx_ref):
    i = pl.program_id(0)
    y = jnp.dot(ef_ref[...], w_ref[...], preferred_element_type=jnp.float32)
    y = y + b_ref[0, 0]
    ee_ref[...] = y
    m = jnp.max(y, axis=0, keepdims=True)

    @pl.when(i == 0)
    def _():
        mx_ref[...] = m

    @pl.when(i > 0)
    def _():
        mx_ref[...] = jnp.maximum(mx_ref[...], m)


def _sc_body(nf, asrc, adst, ee, ef, src, dst, cvec,
             outG, outM,
             sG, sM,
             srcv, dstv, eev, efv, asv, adsv, rowsv, stM, cvv, idxv, idx8v,
             sem):
    c = lax.axis_index("c")
    s = lax.axis_index("s")
    wid = s * NC + c

    z16 = jnp.zeros((16,), jnp.float32)

    def zfill(i, _):
        for j in range(8):
            rowsv[i, pl.ds(j * 16, 16)] = z16
        stM[i, pl.ds(0, 16)] = z16
        stM[i, pl.ds(16, 16)] = z16
        return 0

    lax.fori_loop(0, CH, zfill, 0)

    for k in range(10):
        pltpu.sync_copy(rowsv, sG.at[pl.ds(s * RPT + k * CH, CH)])
        pltpu.sync_copy(stM, sM.at[pl.ds(s * RPT + k * CH, CH)])
    cvv[pl.ds(0, 16)] = jnp.zeros((16,), jnp.float32)

    oh = jnp.where(lax.iota(jnp.int32, 16) == 0, 1.0, 0.0).astype(jnp.float32)
    cv = cvv[...]

    for g in range(CH // 16):
        idxv[pl.ds(g * 16, 16)] = (
            jnp.full((16,), wid * EPW + g * 16, jnp.int32)
            + lax.iota(jnp.int32, 16))
    idx8v[pl.ds(0, 16)] = (
        jnp.full((16,), wid * (EPW // 8), jnp.int32)
        + jnp.minimum(lax.iota(jnp.int32, 16), 7))

    def chunk_body(t):
        pltpu.async_copy(src.at[idxv], srcv, sem).wait()
        return
        pltpu.async_copy(dst.at[idxv], dstv, sem).wait()
        pltpu.async_copy(ee.at[idxv], eev, sem).wait()
        pltpu.async_copy(ef.at[idx8v], efv, sem).wait()
        pltpu.async_copy(asrc.at[srcv], asv, sem).wait()
        pltpu.async_copy(adst.at[dstv], adsv, sem).wait()
        pltpu.async_copy(nf.at[srcv], rowsv, sem).wait()
        for g in range(CH // 16):
            sl = pl.ds(g * 16, 16)
            t_ = asv[sl] + adsv[sl] + eev[sl]
            e = jnp.maximum(t_, 0.01 * t_)
            p16 = jnp.exp(e - cv)
            for l in range(16):
                i = g * 16 + l
                pi = p16[l]
                for j in range(8):
                    rowsv[i, pl.ds(j * 16, 16)] = rowsv[i, pl.ds(j * 16, 16)] * pi
                stM[i, pl.ds(0, 16)] = efv[i // 8, pl.ds((i % 8) * 16, 16)] * pi
                stM[i, pl.ds(16, 16)] = oh * pi
        pass
        for g in range(CH // 16):
            sl = pl.ds(g * 16, 16)
            idxv[sl] = idxv[sl] + CH
        idx8v[pl.ds(0, 16)] = idx8v[pl.ds(0, 16)] + (CH // 8)

    chunk_body(0)
    pltpu.sync_copy(rowsv, outG.at[c, pl.ds(s * CH, CH)])
    return

    row = s * RPT
    pltpu.sync_copy(sG.at[pl.ds(row, RPT + REM)], outG.at[c, pl.ds(row, RPT + REM)])
    pltpu.sync_copy(sM.at[pl.ds(row, RPT + REM)], outM.at[c, pl.ds(row, RPT + REM)])


def _fin_body(nf_ref, g_ref, m_ref, den_ref,
              wne1_ref, wne2_ref, wo1_ref, wo2_ref, out_ref):
    den = den_ref[...]
    den = jnp.where(den > 0.0, den, 1.0)
    z = (jnp.dot(g_ref[...], wne1_ref[...], preferred_element_type=jnp.float32)
         + jnp.dot(m_ref[...], wne2_ref[...], preferred_element_type=jnp.float32))
    z = z / den
    y = (jnp.dot(nf_ref[...], wo1_ref[...], preferred_element_type=jnp.float32)
         + jnp.dot(z, wo2_ref[...], preferred_element_type=jnp.float32))
    out_ref[...] = jnp.maximum(y, 0.0)


def kernel(nfeats, edge_index, efeats, W_ne, W_attn, b_attn, W_out):
    nfeats = jnp.asarray(nfeats, jnp.float32)
    efeats = jnp.asarray(efeats, jnp.float32)

    wa2 = W_attn[:2 * DIN, 0].reshape(2, DIN).T
    wae = W_attn[2 * DIN:, :]
    b2 = b_attn.reshape(1, 1)
    W_ne2p = jnp.concatenate([W_ne[DIN:], jnp.zeros((16, DOUT), jnp.float32)], 0)

    BN = 1000
    ad, mx1 = pl.pallas_call(
        _p1_body,
        grid=(N // BN,),
        in_specs=[pl.BlockSpec((BN, DIN), lambda i: (i, 0)),
                  pl.BlockSpec((DIN, 2), lambda i: (0, 0))],
        out_specs=[pl.BlockSpec((BN, 2), lambda i: (i, 0)),
                   pl.BlockSpec((1, 2), lambda i: (0, 0))],
        out_shape=[jax.ShapeDtypeStruct((N, 2), jnp.float32),
                   jax.ShapeDtypeStruct((1, 2), jnp.float32)],
    )(nfeats, wa2)

    BE = 16000
    ee2, mx2 = pl.pallas_call(
        _p2_body,
        grid=(E // BE,),
        in_specs=[pl.BlockSpec((BE, DE), lambda i: (i, 0)),
                  pl.BlockSpec((DE, 1), lambda i: (0, 0)),
                  pl.BlockSpec((1, 1), lambda i: (0, 0))],
        out_specs=[pl.BlockSpec((BE, 1), lambda i: (i, 0)),
                   pl.BlockSpec((1, 1), lambda i: (0, 0))],
        out_shape=[jax.ShapeDtypeStruct((E, 1), jnp.float32),
                   jax.ShapeDtypeStruct((1, 1), jnp.float32)],
    )(efeats, wae, b2)

    c = jnp.maximum(mx1[0, 0] + mx1[0, 1] + mx2[0, 0], 0.0)
    cvec = jnp.full((16,), c, jnp.float32)

    asrc = ad[:, 0]
    adst = ad[:, 1]
    src = edge_index[0]
    dst = edge_index[1]
    ee = ee2[:, 0]

    logit = jnp.take(asrc, src, axis=0) + jnp.take(adst, dst, axis=0) + ee
    logit = jnp.maximum(logit, 0.01 * logit)
    p = jnp.exp(logit - c)
    den = jax.ops.segment_sum(p, dst, num_segments=N)
    g = jax.ops.segment_sum(p[:, None] * jnp.take(nfeats, src, axis=0), dst,
                            num_segments=N)
    m = jax.ops.segment_sum(p[:, None] * efeats, dst, num_segments=N)

    out = pl.pallas_call(
        _fin_body,
        grid=(N // BN,),
        in_specs=[pl.BlockSpec((BN, DIN), lambda i: (i, 0)),
                  pl.BlockSpec((BN, DIN), lambda i: (i, 0)),
                  pl.BlockSpec((BN, DE), lambda i: (i, 0)),
                  pl.BlockSpec((BN, 1), lambda i: (i, 0)),
                  pl.BlockSpec((DIN, DOUT), lambda i: (0, 0)),
                  pl.BlockSpec((DE, DOUT), lambda i: (0, 0)),
                  pl.BlockSpec((DIN, DOUT), lambda i: (0, 0)),
                  pl.BlockSpec((DOUT, DOUT), lambda i: (0, 0))],
        out_specs=pl.BlockSpec((BN, DOUT), lambda i: (i, 0)),
        out_shape=jax.ShapeDtypeStruct((N, DOUT), jnp.float32),
    )(nfeats, g, m, den[:, None],
      W_ne[:DIN], W_ne[DIN:], W_out[:DIN], W_out[DIN:])
    return out
